# baseline (speedup 1.0000x reference)
import jax
import jax.numpy as jnp
from jax.experimental import pallas as pl
from jax.experimental.pallas import tpu as pltpu

_N_HEADS = 16
_SPAN = (_N_HEADS - 1) // 2
_N_VALS = 2 * _SPAN + 1


def _profile_kernel(w_ref, v2_ref, *, l, width):
    p0 = l - 8
    s = jax.lax.broadcasted_iota(jnp.int32, (8, width), 0)
    p = jax.lax.broadcasted_iota(jnp.int32, (8, width), 1)
    rp = jnp.clip(p - s - p0, 0, _N_VALS - 1)
    acc = jnp.full((8, width), w_ref[0, 0, 0], dtype=jnp.float32)
    for k in range(1, _N_VALS):
        acc = jnp.where(rp == k, w_ref[0, 0, k], acc)
    v2_ref[0, :, :] = acc


def _emit_kernel(v2_ref, o_ref, *, br, l):
    rb = pl.program_id(1)
    i0 = rb * br
    width = 2 * l
    v2 = v2_ref[0, :, :]

    def body(g, _):
        off = (l - 1) - i0 - 8 * g
        rolled = pltpu.roll(v2, width - off, axis=1)
        o_ref[0, 0, pl.ds(8 * g, 8), :] = rolled[:, :l]
        return 0

    jax.lax.fori_loop(0, br // 8, body, 0, unroll=True)


def kernel(attn, W):
    n_heads = attn.shape[0]
    l = attn.shape[2]
    br = min(256, l)
    width = 2 * l
    wt = W.T.reshape(n_heads, 1, n_heads).astype(jnp.float32)
    v2 = pl.pallas_call(
        lambda w_ref, v2_ref: _profile_kernel(w_ref, v2_ref, l=l, width=width),
        grid=(n_heads,),
        in_specs=[pl.BlockSpec((1, 1, n_heads), lambda h: (h, 0, 0))],
        out_specs=pl.BlockSpec((1, 8, width), lambda h: (h, 0, 0)),
        out_shape=jax.ShapeDtypeStruct((n_heads, 8, width), jnp.float32),
    )(wt)
    out = pl.pallas_call(
        lambda v2_ref, o_ref: _emit_kernel(v2_ref, o_ref, br=br, l=l),
        grid=(n_heads, l // br),
        in_specs=[pl.BlockSpec((1, 8, width), lambda h, rb: (h, 0, 0))],
        out_specs=pl.BlockSpec((1, 1, br, l), lambda h, rb: (h, 0, rb, 0)),
        out_shape=jax.ShapeDtypeStruct((n_heads, 1, l, l), jnp.float32),
    )(v2)
    return out


# speedup vs baseline: 1.2287x; 1.2287x over previous
import jax
import jax.numpy as jnp
from jax.experimental import pallas as pl
from jax.experimental.pallas import tpu as pltpu

_N_HEADS = 16
_SPAN = (_N_HEADS - 1) // 2
_N_VALS = 2 * _SPAN + 1


def _profile_kernel(w_ref, v2_ref, *, l, width):
    p0 = l - 8
    s = jax.lax.broadcasted_iota(jnp.int32, (8, width), 0)
    p = jax.lax.broadcasted_iota(jnp.int32, (8, width), 1)
    rp = jnp.clip(p - s - p0, 0, _N_VALS - 1)
    acc = jnp.full((8, width), w_ref[0, 0, 0], dtype=jnp.float32)
    for k in range(1, _N_VALS):
        acc = jnp.where(rp == k, w_ref[0, 0, k], acc)
    v2_ref[0, :, :] = acc


def _emit_kernel(v2_ref, o_ref, *, br, l):
    rb = pl.program_id(1)
    i0 = rb * br
    strip = min(l + 128, 2 * l)

    def body(g, _):
        off = (l - 1) - i0 - 8 * g
        q = off // 128
        m = off - q * 128
        aligned = v2_ref[0, :, pl.ds(q * 128, strip)]
        rolled = pltpu.roll(aligned, strip - m, axis=1)
        o_ref[0, 0, pl.ds(8 * g, 8), :] = rolled[:, :l]
        return 0

    jax.lax.fori_loop(0, br // 8, body, 0, unroll=True)


def kernel(attn, W):
    n_heads = attn.shape[0]
    l = attn.shape[2]
    br = min(256, l)
    width = 2 * l
    wt = W.T.reshape(n_heads, 1, n_heads).astype(jnp.float32)
    v2 = pl.pallas_call(
        lambda w_ref, v2_ref: _profile_kernel(w_ref, v2_ref, l=l, width=width),
        grid=(n_heads,),
        in_specs=[pl.BlockSpec((1, 1, n_heads), lambda h: (h, 0, 0))],
        out_specs=pl.BlockSpec((1, 8, width), lambda h: (h, 0, 0)),
        out_shape=jax.ShapeDtypeStruct((n_heads, 8, width), jnp.float32),
    )(wt)
    out = pl.pallas_call(
        lambda v2_ref, o_ref: _emit_kernel(v2_ref, o_ref, br=br, l=l),
        grid=(n_heads, l // br),
        in_specs=[pl.BlockSpec((1, 8, width), lambda h, rb: (h, 0, 0))],
        out_specs=pl.BlockSpec((1, 1, br, l), lambda h, rb: (h, 0, rb, 0)),
        out_shape=jax.ShapeDtypeStruct((n_heads, 1, l, l), jnp.float32),
    )(v2)
    return out


# speedup vs baseline: 1.4989x; 1.2199x over previous
import jax
import jax.numpy as jnp
from jax.experimental import pallas as pl
from jax.experimental.pallas import tpu as pltpu

_N_HEADS = 16
_SPAN = (_N_HEADS - 1) // 2
_N_VALS = 2 * _SPAN + 1


def _profile_kernel(w_ref, v2_ref, *, l, width):
    p0 = l - 8
    s = jax.lax.broadcasted_iota(jnp.int32, (8, width), 0)
    p = jax.lax.broadcasted_iota(jnp.int32, (8, width), 1)
    rp = jnp.clip(p - s - p0, 0, _N_VALS - 1)
    acc = jnp.full((8, width), w_ref[0, 0, 0], dtype=jnp.float32)
    for k in range(1, _N_VALS):
        acc = jnp.where(rp == k, w_ref[0, 0, k], acc)
    v2_ref[0, :, :] = acc


def _emit_kernel(v2_ref, o_ref, *, br, l):
    rb = pl.program_id(1)
    i0 = rb * br
    strip = min(l + 128, 2 * l)

    def body(g, _):
        off = (l - 1) - i0 - 8 * g
        q = off // 128
        m = off - q * 128
        aligned = v2_ref[0, :, pl.ds(q * 128, strip)]
        rolled = pltpu.roll(aligned, strip - m, axis=1)
        o_ref[0, 0, pl.ds(8 * g, 8), :] = rolled[:, :l]
        return 0

    jax.lax.fori_loop(0, br // 8, body, 0, unroll=True)


def kernel(attn, W):
    n_heads = attn.shape[0]
    l = attn.shape[2]
    br = min(512, l)
    width = 2 * l
    wt = W.T.reshape(n_heads, 1, n_heads).astype(jnp.float32)
    v2 = pl.pallas_call(
        lambda w_ref, v2_ref: _profile_kernel(w_ref, v2_ref, l=l, width=width),
        grid=(n_heads,),
        in_specs=[pl.BlockSpec((1, 1, n_heads), lambda h: (h, 0, 0))],
        out_specs=pl.BlockSpec((1, 8, width), lambda h: (h, 0, 0)),
        out_shape=jax.ShapeDtypeStruct((n_heads, 8, width), jnp.float32),
    )(wt)
    out = pl.pallas_call(
        lambda v2_ref, o_ref: _emit_kernel(v2_ref, o_ref, br=br, l=l),
        grid=(n_heads, l // br),
        in_specs=[pl.BlockSpec((1, 8, width), lambda h, rb: (h, 0, 0))],
        out_specs=pl.BlockSpec((1, 1, br, l), lambda h, rb: (h, 0, rb, 0)),
        out_shape=jax.ShapeDtypeStruct((n_heads, 1, l, l), jnp.float32),
    )(v2)
    return out


# speedup vs baseline: 1.6575x; 1.1058x over previous
import jax
import jax.numpy as jnp
from jax.experimental import pallas as pl
from jax.experimental.pallas import tpu as pltpu

_N_HEADS = 16
_SPAN = (_N_HEADS - 1) // 2
_N_VALS = 2 * _SPAN + 1


def _profile_kernel(w_ref, v2_ref, *, l, width):
    p0 = l - 8
    s = jax.lax.broadcasted_iota(jnp.int32, (8, width), 0)
    p = jax.lax.broadcasted_iota(jnp.int32, (8, width), 1)
    rp = jnp.clip(p - s - p0, 0, _N_VALS - 1)
    acc = jnp.full((8, width), w_ref[0, 0, 0], dtype=jnp.float32)
    for k in range(1, _N_VALS):
        acc = jnp.where(rp == k, w_ref[0, 0, k], acc)
    v2_ref[0, :, :] = acc


def _emit_kernel(v2_ref, o_ref, *, br, l):
    rb = pl.program_id(1)
    i0 = rb * br
    strip = min(l + 128, 2 * l)

    def body(g, _):
        off = (l - 1) - i0 - 8 * g
        q = off // 128
        m = off - q * 128
        aligned = v2_ref[0, :, pl.ds(q * 128, strip)]
        rolled = pltpu.roll(aligned, strip - m, axis=1)
        o_ref[0, 0, pl.ds(8 * g, 8), :] = rolled[:, :l]
        return 0

    jax.lax.fori_loop(0, br // 8, body, 0, unroll=True)


def kernel(attn, W):
    n_heads = attn.shape[0]
    l = attn.shape[2]
    br = min(1024, l)
    width = 2 * l
    wt = W.T.reshape(n_heads, 1, n_heads).astype(jnp.float32)
    v2 = pl.pallas_call(
        lambda w_ref, v2_ref: _profile_kernel(w_ref, v2_ref, l=l, width=width),
        grid=(n_heads,),
        in_specs=[pl.BlockSpec((1, 1, n_heads), lambda h: (h, 0, 0))],
        out_specs=pl.BlockSpec((1, 8, width), lambda h: (h, 0, 0)),
        out_shape=jax.ShapeDtypeStruct((n_heads, 8, width), jnp.float32),
    )(wt)
    out = pl.pallas_call(
        lambda v2_ref, o_ref: _emit_kernel(v2_ref, o_ref, br=br, l=l),
        grid=(n_heads, l // br),
        in_specs=[pl.BlockSpec((1, 8, width), lambda h, rb: (h, 0, 0))],
        out_specs=pl.BlockSpec((1, 1, br, l), lambda h, rb: (h, 0, rb, 0)),
        out_shape=jax.ShapeDtypeStruct((n_heads, 1, l, l), jnp.float32),
    )(v2)
    return out


# speedup vs baseline: 1.7164x; 1.0356x over previous
import jax
import jax.numpy as jnp
from jax.experimental import pallas as pl
from jax.experimental.pallas import tpu as pltpu

_N_HEADS = 16
_SPAN = (_N_HEADS - 1) // 2
_N_VALS = 2 * _SPAN + 1


def _profile_kernel(w_ref, v2_ref, *, l, width):
    p0 = l - 8
    s = jax.lax.broadcasted_iota(jnp.int32, (8, width), 0)
    p = jax.lax.broadcasted_iota(jnp.int32, (8, width), 1)
    rp = jnp.clip(p - s - p0, 0, _N_VALS - 1)
    acc = jnp.full((8, width), w_ref[0, 0, 0], dtype=jnp.float32)
    for k in range(1, _N_VALS):
        acc = jnp.where(rp == k, w_ref[0, 0, k], acc)
    v2_ref[0, :, :] = acc


def _emit_kernel(v2_ref, o_ref, *, br, l):
    rb = pl.program_id(1)
    i0 = rb * br
    strip = min(l + 128, 2 * l)

    def body(g, _):
        off = (l - 1) - i0 - 8 * g
        q = off // 128
        m = off - q * 128
        aligned = v2_ref[0, :, pl.ds(q * 128, strip)]
        rolled = pltpu.roll(aligned, strip - m, axis=1)
        o_ref[0, 0, pl.ds(8 * g, 8), :] = rolled[:, :l]
        return 0

    jax.lax.fori_loop(0, br // 8, body, 0, unroll=True)


def kernel(attn, W):
    n_heads = attn.shape[0]
    l = attn.shape[2]
    br = min(2048, l)
    width = 2 * l
    wt = W.T.reshape(n_heads, 1, n_heads).astype(jnp.float32)
    v2 = pl.pallas_call(
        lambda w_ref, v2_ref: _profile_kernel(w_ref, v2_ref, l=l, width=width),
        grid=(n_heads,),
        in_specs=[pl.BlockSpec((1, 1, n_heads), lambda h: (h, 0, 0))],
        out_specs=pl.BlockSpec((1, 8, width), lambda h: (h, 0, 0)),
        out_shape=jax.ShapeDtypeStruct((n_heads, 8, width), jnp.float32),
    )(wt)
    out = pl.pallas_call(
        lambda v2_ref, o_ref: _emit_kernel(v2_ref, o_ref, br=br, l=l),
        grid=(n_heads, l // br),
        in_specs=[pl.BlockSpec((1, 8, width), lambda h, rb: (h, 0, 0))],
        out_specs=pl.BlockSpec((1, 1, br, l), lambda h, rb: (h, 0, rb, 0)),
        out_shape=jax.ShapeDtypeStruct((n_heads, 1, l, l), jnp.float32),
    )(v2)
    return out
